# baseline (speedup 1.0000x reference)
import functools

import jax
import jax.numpy as jnp
from jax.experimental import pallas as pl
from jax.experimental.pallas import tpu as pltpu
from jax.experimental.pallas import tpu_sc as plsc
from jax import lax

B, S, HID, FFN, E, TOPK = 2, 2048, 1024, 4096, 8, 2
T = B * S
A = T * TOPK

BM = 128
NBLK = A // BM + E
NROWS = NBLK * BM
FFN_BLK = 1024
FH = FFN // 2
TOK_BLK = 512


def _router_body(x_ref, gw_ref, logits_ref, w_ref, ids_ref):
    xb = x_ref[...]
    logits = jnp.dot(xb, gw_ref[...], preferred_element_type=jnp.float32)
    logits_ref[...] = logits
    m = jnp.max(logits, axis=1, keepdims=True)
    p = jnp.exp(logits - m)
    p = p / jnp.sum(p, axis=1, keepdims=True)
    e_iota = jax.lax.broadcasted_iota(jnp.int32, (TOK_BLK, E), 1)
    a1 = jnp.argmax(p, axis=1)
    oh1 = (e_iota == a1[:, None])
    p1 = jnp.max(p, axis=1)
    p_masked = jnp.where(oh1, -jnp.inf, p)
    a2 = jnp.argmax(p_masked, axis=1)
    p2 = jnp.max(p_masked, axis=1)
    denom = p1 + p2
    w_ref[...] = jnp.stack([p1 / denom, p2 / denom], axis=1)
    ids_ref[...] = jnp.stack([a1, a2], axis=1).astype(jnp.int32)


def _router(x, gate_w):
    grid = (T // TOK_BLK,)
    return pl.pallas_call(
        _router_body,
        grid=grid,
        in_specs=[
            pl.BlockSpec((TOK_BLK, HID), lambda b: (b, 0)),
            pl.BlockSpec((HID, E), lambda b: (0, 0)),
        ],
        out_specs=[
            pl.BlockSpec((TOK_BLK, E), lambda b: (b, 0)),
            pl.BlockSpec((TOK_BLK, TOPK), lambda b: (b, 0)),
            pl.BlockSpec((TOK_BLK, TOPK), lambda b: (b, 0)),
        ],
        out_shape=[
            jax.ShapeDtypeStruct((T, E), jnp.float32),
            jax.ShapeDtypeStruct((T, TOPK), jnp.float32),
            jax.ShapeDtypeStruct((T, TOPK), jnp.int32),
        ],
    )(x, gate_w)


def _swiglu_acc(xb, w1_ref, w3_ref, w2_ref):
    acc = jnp.zeros((BM, HID), jnp.float32)
    for f in range(FH // FFN_BLK):
        w1t = w1_ref[0, :, f * FFN_BLK:(f + 1) * FFN_BLK]
        w3t = w3_ref[0, :, f * FFN_BLK:(f + 1) * FFN_BLK]
        w2t = w2_ref[0, f * FFN_BLK:(f + 1) * FFN_BLK, :]
        a = jnp.dot(xb, w1t, preferred_element_type=jnp.float32)
        g = jnp.dot(xb, w3t, preferred_element_type=jnp.float32)
        h = a * jax.nn.sigmoid(a) * g
        acc += jnp.dot(h, w2t, preferred_element_type=jnp.float32)
    return acc


def _gffn1_body(be_ref, xg_ref, w1_ref, w3_ref, w2_ref, out_ref):
    out_ref[...] = _swiglu_acc(xg_ref[...], w1_ref, w3_ref, w2_ref)


def _gffn2_body(be_ref, xg_ref, w1_ref, w3_ref, w2_ref, prev_ref, wr_ref,
                out_ref):
    acc = _swiglu_acc(xg_ref[...], w1_ref, w3_ref, w2_ref)
    out_ref[...] = (prev_ref[...] + acc) * wr_ref[...]


def _gffn_half1(xg, block_expert, W1, W3, W2):
    grid_spec = pltpu.PrefetchScalarGridSpec(
        num_scalar_prefetch=1,
        grid=(NBLK,),
        in_specs=[
            pl.BlockSpec((BM, HID), lambda b, be: (b, 0)),
            pl.BlockSpec((1, HID, FH), lambda b, be: (be[b], 0, 0)),
            pl.BlockSpec((1, HID, FH), lambda b, be: (be[b], 0, 0)),
            pl.BlockSpec((1, FH, HID), lambda b, be: (be[b], 0, 0)),
        ],
        out_specs=pl.BlockSpec((BM, HID), lambda b, be: (b, 0)),
    )
    return pl.pallas_call(
        _gffn1_body,
        grid_spec=grid_spec,
        out_shape=jax.ShapeDtypeStruct((NROWS, HID), jnp.float32),
        compiler_params=pltpu.CompilerParams(
            dimension_semantics=("arbitrary",),
            vmem_limit_bytes=63 * 1024 * 1024,
        ),
    )(block_expert, xg, W1, W3, W2)


def _gffn_half2(xg, block_expert, W1, W3, W2, prev, wrow):
    grid_spec = pltpu.PrefetchScalarGridSpec(
        num_scalar_prefetch=1,
        grid=(NBLK,),
        in_specs=[
            pl.BlockSpec((BM, HID), lambda b, be: (b, 0)),
            pl.BlockSpec((1, HID, FH), lambda b, be: (be[b], 0, 1)),
            pl.BlockSpec((1, HID, FH), lambda b, be: (be[b], 0, 1)),
            pl.BlockSpec((1, FH, HID), lambda b, be: (be[b], 1, 0)),
            pl.BlockSpec((BM, HID), lambda b, be: (b, 0)),
            pl.BlockSpec((BM, 1), lambda b, be: (b, 0)),
        ],
        out_specs=pl.BlockSpec((BM, HID), lambda b, be: (b, 0)),
    )
    return pl.pallas_call(
        _gffn2_body,
        grid_spec=grid_spec,
        out_shape=jax.ShapeDtypeStruct((NROWS, HID), jnp.float32),
        compiler_params=pltpu.CompilerParams(
            dimension_semantics=("arbitrary",),
            vmem_limit_bytes=63 * 1024 * 1024,
        ),
    )(block_expert, xg, W1, W3, W2, prev, wrow)


NC, NS = 2, 16
NW = NC * NS
TPW = T // NW
CH = 32

_sc_mesh = plsc.VectorSubcoreMesh(core_axis_name="c", subcore_axis_name="s")


@functools.partial(
    pl.kernel, mesh=_sc_mesh,
    out_type=jax.ShapeDtypeStruct((T, HID), jnp.float32),
    scratch_types=[
        pltpu.VMEM((CH,), jnp.int32),
        pltpu.VMEM((CH,), jnp.int32),
        pltpu.VMEM((CH, HID), jnp.float32),
        pltpu.VMEM((CH, HID), jnp.float32),
        pltpu.SemaphoreType.DMA,
        pltpu.SemaphoreType.DMA,
    ],
)
def _sc_combine(og_hbm, i0_hbm, i1_hbm, out_hbm,
                i0_v, i1_v, r0_v, r1_v, s0, s1):
    wid = lax.axis_index("s") * NC + lax.axis_index("c")
    base = wid * TPW
    for c in range(TPW // CH):
        off = base + c * CH
        pltpu.sync_copy(i0_hbm.at[pl.ds(off, CH)], i0_v)
        pltpu.sync_copy(i1_hbm.at[pl.ds(off, CH)], i1_v)
        cp0 = pltpu.async_copy(og_hbm.at[i0_v], r0_v, s0)
        cp1 = pltpu.async_copy(og_hbm.at[i1_v], r1_v, s1)
        cp0.wait()
        cp1.wait()
        for r in range(CH):
            def _body(l, carry, r=r):
                sl = pl.ds(l * 16, 16)
                r0_v[r, sl] = r0_v[r, sl] + r1_v[r, sl]
                return carry
            lax.fori_loop(0, HID // 16, _body, 0)
        pltpu.sync_copy(r0_v, out_hbm.at[pl.ds(off, CH)])


@functools.partial(jax.jit, static_argnames=())
def kernel(hidden_states, gate_w, W1, W3, W2):
    b, s, hid = hidden_states.shape
    x = hidden_states.reshape(-1, hid)
    router_logits, w, ids = _router(x, gate_w)

    flat_ids = ids.reshape(-1)
    oh = (flat_ids[:, None] == jnp.arange(E, dtype=jnp.int32)[None, :])
    counts = jnp.sum(oh.astype(jnp.int32), axis=0)
    nblk_e = (counts + BM - 1) // BM
    blk_end = jnp.cumsum(nblk_e)
    pstart = (blk_end - nblk_e) * BM
    csum = jnp.cumsum(oh.astype(jnp.int32), axis=0)
    rank = jnp.take_along_axis(csum, flat_ids[:, None], axis=1)[:, 0] - 1
    pos = pstart[flat_ids] + rank
    blk_iota = jnp.arange(NBLK, dtype=jnp.int32)
    block_expert = jnp.sum(
        (blk_iota[:, None] >= blk_end[None, :]).astype(jnp.int32), axis=1)
    block_expert = jnp.minimum(block_expert, E - 1)
    src = jnp.zeros((NROWS,), jnp.int32).at[pos].set(
        jnp.arange(A, dtype=jnp.int32) // TOPK)
    wrow = jnp.zeros((NROWS,), jnp.float32).at[pos].set(
        w.reshape(-1)).reshape(NROWS, 1)

    xg = x[src]
    og0 = _gffn_half1(xg, block_expert, W1, W3, W2)
    og = _gffn_half2(xg, block_expert, W1, W3, W2, og0, wrow)
    pos2 = pos.reshape(T, TOPK)
    i0 = pos2[:, 0] + 0
    i1 = pos2[:, 1] + 0
    final = _sc_combine(og, i0, i1)
    return final.reshape(b, s, hid), router_logits


# speedup vs baseline: 1.0169x; 1.0169x over previous
import functools

import jax
import jax.numpy as jnp
from jax.experimental import pallas as pl
from jax.experimental.pallas import tpu as pltpu
from jax.experimental.pallas import tpu_sc as plsc
from jax import lax

B, S, HID, FFN, E, TOPK = 2, 2048, 1024, 4096, 8, 2
T = B * S
A = T * TOPK

BM = 256
NBLK = A // BM + E
NROWS = NBLK * BM
FFN_BLK = 2048
FH = FFN // 2
TOK_BLK = 512


def _router_body(x_ref, gw_ref, logits_ref, w_ref, ids_ref):
    xb = x_ref[...]
    logits = jnp.dot(xb, gw_ref[...], preferred_element_type=jnp.float32)
    logits_ref[...] = logits
    m = jnp.max(logits, axis=1, keepdims=True)
    p = jnp.exp(logits - m)
    p = p / jnp.sum(p, axis=1, keepdims=True)
    e_iota = jax.lax.broadcasted_iota(jnp.int32, (TOK_BLK, E), 1)
    a1 = jnp.argmax(p, axis=1)
    oh1 = (e_iota == a1[:, None])
    p1 = jnp.max(p, axis=1)
    p_masked = jnp.where(oh1, -jnp.inf, p)
    a2 = jnp.argmax(p_masked, axis=1)
    p2 = jnp.max(p_masked, axis=1)
    denom = p1 + p2
    w_ref[...] = jnp.stack([p1 / denom, p2 / denom], axis=1)
    ids_ref[...] = jnp.stack([a1, a2], axis=1).astype(jnp.int32)


def _router(x, gate_w):
    grid = (T // TOK_BLK,)
    return pl.pallas_call(
        _router_body,
        grid=grid,
        in_specs=[
            pl.BlockSpec((TOK_BLK, HID), lambda b: (b, 0)),
            pl.BlockSpec((HID, E), lambda b: (0, 0)),
        ],
        out_specs=[
            pl.BlockSpec((TOK_BLK, E), lambda b: (b, 0)),
            pl.BlockSpec((TOK_BLK, TOPK), lambda b: (b, 0)),
            pl.BlockSpec((TOK_BLK, TOPK), lambda b: (b, 0)),
        ],
        out_shape=[
            jax.ShapeDtypeStruct((T, E), jnp.float32),
            jax.ShapeDtypeStruct((T, TOPK), jnp.float32),
            jax.ShapeDtypeStruct((T, TOPK), jnp.int32),
        ],
    )(x, gate_w)


def _swiglu_acc(xb, w1_ref, w3_ref, w2_ref):
    acc = jnp.zeros((BM, HID), jnp.float32)
    for f in range(FH // FFN_BLK):
        w1t = w1_ref[0, :, f * FFN_BLK:(f + 1) * FFN_BLK]
        w3t = w3_ref[0, :, f * FFN_BLK:(f + 1) * FFN_BLK]
        w2t = w2_ref[0, f * FFN_BLK:(f + 1) * FFN_BLK, :]
        a = jnp.dot(xb, w1t, preferred_element_type=jnp.float32)
        g = jnp.dot(xb, w3t, preferred_element_type=jnp.float32)
        h = a * jax.nn.sigmoid(a) * g
        acc += jnp.dot(h, w2t, preferred_element_type=jnp.float32)
    return acc


def _gffn1_body(be_ref, xg_ref, w1_ref, w3_ref, w2_ref, out_ref):
    out_ref[...] = _swiglu_acc(xg_ref[...], w1_ref, w3_ref, w2_ref)


def _gffn2_body(be_ref, xg_ref, w1_ref, w3_ref, w2_ref, prev_ref, wr_ref,
                out_ref):
    acc = _swiglu_acc(xg_ref[...], w1_ref, w3_ref, w2_ref)
    out_ref[...] = (prev_ref[...] + acc) * wr_ref[...]


def _gffn_half1(xg, block_expert, W1, W3, W2):
    grid_spec = pltpu.PrefetchScalarGridSpec(
        num_scalar_prefetch=1,
        grid=(NBLK,),
        in_specs=[
            pl.BlockSpec((BM, HID), lambda b, be: (b, 0)),
            pl.BlockSpec((1, HID, FH), lambda b, be: (be[b], 0, 0)),
            pl.BlockSpec((1, HID, FH), lambda b, be: (be[b], 0, 0)),
            pl.BlockSpec((1, FH, HID), lambda b, be: (be[b], 0, 0)),
        ],
        out_specs=pl.BlockSpec((BM, HID), lambda b, be: (b, 0)),
    )
    return pl.pallas_call(
        _gffn1_body,
        grid_spec=grid_spec,
        out_shape=jax.ShapeDtypeStruct((NROWS, HID), jnp.float32),
        compiler_params=pltpu.CompilerParams(
            dimension_semantics=("arbitrary",),
            vmem_limit_bytes=63 * 1024 * 1024,
        ),
    )(block_expert, xg, W1, W3, W2)


def _gffn_half2(xg, block_expert, W1, W3, W2, prev, wrow):
    grid_spec = pltpu.PrefetchScalarGridSpec(
        num_scalar_prefetch=1,
        grid=(NBLK,),
        in_specs=[
            pl.BlockSpec((BM, HID), lambda b, be: (b, 0)),
            pl.BlockSpec((1, HID, FH), lambda b, be: (be[b], 0, 1)),
            pl.BlockSpec((1, HID, FH), lambda b, be: (be[b], 0, 1)),
            pl.BlockSpec((1, FH, HID), lambda b, be: (be[b], 1, 0)),
            pl.BlockSpec((BM, HID), lambda b, be: (b, 0)),
            pl.BlockSpec((BM, 1), lambda b, be: (b, 0)),
        ],
        out_specs=pl.BlockSpec((BM, HID), lambda b, be: (b, 0)),
    )
    return pl.pallas_call(
        _gffn2_body,
        grid_spec=grid_spec,
        out_shape=jax.ShapeDtypeStruct((NROWS, HID), jnp.float32),
        compiler_params=pltpu.CompilerParams(
            dimension_semantics=("arbitrary",),
            vmem_limit_bytes=63 * 1024 * 1024,
        ),
    )(block_expert, xg, W1, W3, W2, prev, wrow)


NC, NS = 2, 16
NW = NC * NS
TPW = T // NW
CH = 32

_sc_mesh = plsc.VectorSubcoreMesh(core_axis_name="c", subcore_axis_name="s")


@functools.partial(
    pl.kernel, mesh=_sc_mesh,
    out_type=jax.ShapeDtypeStruct((T, HID), jnp.float32),
    scratch_types=[
        pltpu.VMEM((CH,), jnp.int32),
        pltpu.VMEM((CH,), jnp.int32),
        pltpu.VMEM((CH, HID), jnp.float32),
        pltpu.VMEM((CH, HID), jnp.float32),
        pltpu.SemaphoreType.DMA,
        pltpu.SemaphoreType.DMA,
    ],
)
def _sc_combine(og_hbm, i0_hbm, i1_hbm, out_hbm,
                i0_v, i1_v, r0_v, r1_v, s0, s1):
    wid = lax.axis_index("s") * NC + lax.axis_index("c")
    base = wid * TPW
    for c in range(TPW // CH):
        off = base + c * CH
        pltpu.sync_copy(i0_hbm.at[pl.ds(off, CH)], i0_v)
        pltpu.sync_copy(i1_hbm.at[pl.ds(off, CH)], i1_v)
        cp0 = pltpu.async_copy(og_hbm.at[i0_v], r0_v, s0)
        cp1 = pltpu.async_copy(og_hbm.at[i1_v], r1_v, s1)
        cp0.wait()
        cp1.wait()
        for r in range(CH):
            def _body(l, carry, r=r):
                sl = pl.ds(l * 16, 16)
                r0_v[r, sl] = r0_v[r, sl] + r1_v[r, sl]
                return carry
            lax.fori_loop(0, HID // 16, _body, 0)
        pltpu.sync_copy(r0_v, out_hbm.at[pl.ds(off, CH)])


@functools.partial(jax.jit, static_argnames=())
def kernel(hidden_states, gate_w, W1, W3, W2):
    b, s, hid = hidden_states.shape
    x = hidden_states.reshape(-1, hid)
    router_logits, w, ids = _router(x, gate_w)

    flat_ids = ids.reshape(-1)
    oh = (flat_ids[:, None] == jnp.arange(E, dtype=jnp.int32)[None, :])
    counts = jnp.sum(oh.astype(jnp.int32), axis=0)
    nblk_e = (counts + BM - 1) // BM
    blk_end = jnp.cumsum(nblk_e)
    pstart = (blk_end - nblk_e) * BM
    csum = jnp.cumsum(oh.astype(jnp.int32), axis=0)
    rank = jnp.take_along_axis(csum, flat_ids[:, None], axis=1)[:, 0] - 1
    pos = pstart[flat_ids] + rank
    blk_iota = jnp.arange(NBLK, dtype=jnp.int32)
    block_expert = jnp.sum(
        (blk_iota[:, None] >= blk_end[None, :]).astype(jnp.int32), axis=1)
    block_expert = jnp.minimum(block_expert, E - 1)
    src = jnp.zeros((NROWS,), jnp.int32).at[pos].set(
        jnp.arange(A, dtype=jnp.int32) // TOPK)
    wrow = jnp.zeros((NROWS,), jnp.float32).at[pos].set(
        w.reshape(-1)).reshape(NROWS, 1)

    xg = x[src]
    og0 = _gffn_half1(xg, block_expert, W1, W3, W2)
    og = _gffn_half2(xg, block_expert, W1, W3, W2, og0, wrow)
    pos2 = pos.reshape(T, TOPK)
    i0 = pos2[:, 0] + 0
    i1 = pos2[:, 1] + 0
    final = _sc_combine(og, i0, i1)
    return final.reshape(b, s, hid), router_logits
